# baseline (speedup 1.0000x reference)
import functools

import jax
import jax.numpy as jnp
from jax import lax
from jax.experimental import pallas as pl
from jax.experimental.pallas import tpu as pltpu
from jax.experimental.pallas import tpu_sc as plsc

N_TOK = 16384
N_EMB = 8192
DIM = 64

TN = 512
TK = 2944
K_SC = 2304
K_TC = N_EMB - K_SC
NB = N_TOK // TN
KB = K_TC // TK

CH = 16
NVROW = K_SC // 16

_KS = (0, 42, 0x1BD11BDA ^ 42)
_ROT = ((13, 15, 26, 6), (17, 29, 16, 24))


def _threefry_bits(w):
    one = lambda c: jnp.asarray(c & 0xFFFFFFFF, w.dtype)
    x0 = w
    x1 = (lax.shift_left(w, one(13)) | lax.shift_right_logical(w, one(19))) ^ w
    for r in (15, 26, 6):
        x0 = x0 + x1
        x1 = (lax.shift_left(x1, one(r))
              | lax.shift_right_logical(x1, one(32 - r))) ^ x0
    x0 = x0 + one(42)
    x1 = x1 + one(_KS[2] + 1)
    for b in range(1, 5):
        for r in _ROT[b % 2]:
            x0 = x0 + x1
            x1 = (lax.shift_left(x1, one(r))
                  | lax.shift_right_logical(x1, one(32 - r))) ^ x0
        kx = _KS[(b + 1) % 3]
        if kx:
            x0 = x0 + one(kx)
        x1 = x1 + one(_KS[(b + 2) % 3] + b + 1)
    return x0 ^ x1


def _bits_to_gumbel(bits):
    bits = bits.astype(jnp.uint32)
    fb = (bits >> jnp.uint32(9)) | jnp.uint32(0x3F800000)
    f = lax.bitcast_convert_type(fb, jnp.float32) - jnp.float32(1.0)
    u = jnp.maximum(jnp.float32(1e-10), f + jnp.float32(1e-10))
    return -jnp.log(-jnp.log(u))


def _sc_bits():
    info = plsc.get_sparse_core_info()
    nw = info.num_cores * info.num_subcores
    t_per_w = N_TOK // nw
    n_chunks = t_per_w // CH
    mesh = plsc.VectorSubcoreMesh(core_axis_name="c", subcore_axis_name="s")

    @functools.partial(
        pl.kernel, mesh=mesh,
        compiler_params=pltpu.CompilerParams(use_tc_tiling_on_sc=True),
        out_type=jax.ShapeDtypeStruct((N_TOK, K_SC), jnp.int32),
        scratch_types=[
            pltpu.VMEM((CH, K_SC), jnp.int32),
            pltpu.VMEM((CH, K_SC), jnp.int32),
            pltpu.SemaphoreType.DMA,
            pltpu.SemaphoreType.DMA,
        ],
    )
    def k(bits_hbm, buf0, buf1, sem0, sem1):
        wid = lax.axis_index("s") * info.num_cores + lax.axis_index("c")
        t0 = wid * t_per_w
        lane = lax.iota(jnp.int32, 16)
        bufs = (buf0, buf1)
        sems = (sem0, sem1)

        def fill(c, buf):
            row0 = t0 + c * CH

            def row_body(r, _):
                base_r = (row0 + r) * N_EMB + K_TC + 42

                def vec(v8, _):
                    for u in range(8):
                        c16 = v8 * 8 + u
                        buf[r, pl.ds(c16 * 16, 16)] = _threefry_bits(
                            base_r + c16 * 16 + lane)
                    return 0

                lax.fori_loop(0, NVROW // 8, vec, 0)
                return 0

            lax.fori_loop(0, CH, row_body, 0)

        def fire(c, b):
            fill(c, bufs[b])
            pltpu.async_copy(
                bufs[b], bits_hbm.at[pl.ds(t0 + c * CH, CH), :], sems[b])

        def drain(c, b):
            pltpu.make_async_copy(
                bufs[b], bits_hbm.at[pl.ds(t0 + c * CH, CH), :], sems[b]).wait()

        fire(0, 0)
        fire(1, 1)

        def step(g, _):
            c0 = g * 2
            drain(c0 - 2, 0)
            fire(c0, 0)
            drain(c0 - 1, 1)
            fire(c0 + 1, 1)
            return 0

        lax.fori_loop(1, n_chunks // 2, step, 0)
        drain(n_chunks - 2, 0)
        drain(n_chunks - 1, 1)

    return k()


def _cnorm_body(c_ref, out_ref):
    c = c_ref[...]
    nrm = jnp.sqrt(jnp.sum(c * c, axis=-1, keepdims=True))
    out_ref[...] = c / jnp.maximum(nrm, jnp.float32(1e-8))


def _normalize_codebook(codebook):
    return pl.pallas_call(
        _cnorm_body,
        out_shape=jax.ShapeDtypeStruct((N_EMB, DIM), jnp.float32),
    )(codebook)


def _normalize_x(x_ref):
    x = x_ref[...]
    nrm = jnp.sqrt(jnp.sum(x * x, axis=-1, keepdims=True))
    return x / jnp.maximum(nrm, jnp.float32(1e-8))


def _head_body(x_ref, cn_ref, t_ref, bv_ref, bi_ref):
    i = pl.program_id(0)
    xn = _normalize_x(x_ref)
    temp = t_ref[0, 0]

    rows = i * TN + lax.broadcasted_iota(jnp.int32, (TN, TK), 0)
    cols = lax.broadcasted_iota(jnp.int32, (TN, TK), 1)
    w0 = (rows * N_EMB + cols + 42).astype(jnp.uint32)

    def kstep(j, carry):
        bv, bi = carry
        cn = cn_ref[pl.ds(j * TK, TK), :]
        l = lax.dot_general(
            xn, cn, (((1,), (1,)), ((), ())),
            preferred_element_type=jnp.float32,
        ) / temp
        w = w0 + jnp.uint32(TK) * j.astype(jnp.uint32)
        v = l + _bits_to_gumbel(_threefry_bits(w))
        mv = jnp.max(v, axis=1)
        cand = jnp.where(v == mv[:, None], cols, jnp.int32(TK))
        ma = jnp.min(cand, axis=1) + j * TK
        upd = mv > bv
        return jnp.where(upd, mv, bv), jnp.where(upd, ma, bi)

    bv0 = jnp.full((TN,), -jnp.inf, jnp.float32)
    bi0 = jnp.zeros((TN,), jnp.int32)
    bv, bi = lax.fori_loop(0, KB, kstep, (bv0, bi0))
    bv_ref[...] = bv.reshape(1, 1, TN)
    bi_ref[...] = bi.reshape(1, 1, TN)


def _tail_body(x_ref, cn_ref, t_ref, bits_ref, bv_ref, bi_ref, out_ref):
    xn = _normalize_x(x_ref)
    temp = t_ref[0, 0]
    cn = cn_ref[pl.ds(K_TC, K_SC), :]
    l = lax.dot_general(
        xn, cn, (((1,), (1,)), ((), ())),
        preferred_element_type=jnp.float32,
    ) / temp
    v = l + _bits_to_gumbel(bits_ref[...])
    mv = jnp.max(v, axis=1)
    cols = lax.broadcasted_iota(jnp.int32, (TN, K_SC), 1)
    cand = jnp.where(v == mv[:, None], cols, jnp.int32(K_SC))
    ma = jnp.min(cand, axis=1) + K_TC
    bv = bv_ref[...].reshape(TN)
    bi = bi_ref[...].reshape(TN)
    upd = mv > bv
    out_ref[...] = jnp.where(upd, ma, bi).reshape(1, 1, TN)


def _hard_indices(latent, cn, temperature, bits):
    t2 = temperature.reshape(1, 1)
    bv, bi = pl.pallas_call(
        _head_body,
        grid=(NB,),
        in_specs=[
            pl.BlockSpec((TN, DIM), lambda i: (i, 0)),
            pl.BlockSpec((N_EMB, DIM), lambda i: (0, 0)),
            pl.BlockSpec(memory_space=pltpu.SMEM),
        ],
        out_specs=[
            pl.BlockSpec((1, 1, TN), lambda i: (i, 0, 0)),
            pl.BlockSpec((1, 1, TN), lambda i: (i, 0, 0)),
        ],
        out_shape=[
            jax.ShapeDtypeStruct((NB, 1, TN), jnp.float32),
            jax.ShapeDtypeStruct((NB, 1, TN), jnp.int32),
        ],
    )(latent, cn, t2)
    out = pl.pallas_call(
        _tail_body,
        grid=(NB,),
        in_specs=[
            pl.BlockSpec((TN, DIM), lambda i: (i, 0)),
            pl.BlockSpec((N_EMB, DIM), lambda i: (0, 0)),
            pl.BlockSpec(memory_space=pltpu.SMEM),
            pl.BlockSpec((TN, K_SC), lambda i: (i, 0)),
            pl.BlockSpec((1, 1, TN), lambda i: (i, 0, 0)),
            pl.BlockSpec((1, 1, TN), lambda i: (i, 0, 0)),
        ],
        out_specs=pl.BlockSpec((1, 1, TN), lambda i: (i, 0, 0)),
        out_shape=jax.ShapeDtypeStruct((NB, 1, TN), jnp.int32),
    )(latent, cn, t2, bits, bv, bi)
    return out.reshape(N_TOK)


def _sc_gather(codebook, idx):
    info = plsc.get_sparse_core_info()
    nw = info.num_cores * info.num_subcores
    b_per_w = N_TOK // nw
    mesh = plsc.VectorSubcoreMesh(core_axis_name="c", subcore_axis_name="s")

    @functools.partial(
        pl.kernel, mesh=mesh,
        compiler_params=pltpu.CompilerParams(use_tc_tiling_on_sc=False),
        out_type=jax.ShapeDtypeStruct((N_TOK, DIM), jnp.float32),
        scratch_types=[
            pltpu.VMEM((b_per_w,), jnp.int32),
            pltpu.VMEM((b_per_w, DIM), jnp.float32),
            pltpu.SemaphoreType.DMA,
        ],
    )
    def k(table_hbm, idx_hbm, out_hbm, idx_v, rows_v, sem):
        wid = lax.axis_index("s") * info.num_cores + lax.axis_index("c")
        base = wid * b_per_w
        pltpu.sync_copy(idx_hbm.at[pl.ds(base, b_per_w)], idx_v)
        pltpu.async_copy(table_hbm.at[idx_v], rows_v, sem).wait()
        pltpu.sync_copy(rows_v, out_hbm.at[pl.ds(base, b_per_w)])

    return k(codebook, idx)


def kernel(latent_representation, codebook, temperature):
    bits = _sc_bits()
    cn = _normalize_codebook(codebook)
    idx = _hard_indices(latent_representation, cn, temperature, bits)
    return _sc_gather(codebook, idx)


# speedup vs baseline: 1.0214x; 1.0214x over previous
import functools

import jax
import jax.numpy as jnp
from jax import lax
from jax.experimental import pallas as pl
from jax.experimental.pallas import tpu as pltpu
from jax.experimental.pallas import tpu_sc as plsc

N_TOK = 16384
N_EMB = 8192
DIM = 64

TN = 512
TK = 1920
K_SC = 2432
K_TC = N_EMB - K_SC
NB = N_TOK // TN
KB = K_TC // TK

CH = 16
NVROW = K_SC // 16

_KS = (0, 42, 0x1BD11BDA ^ 42)
_ROT = ((13, 15, 26, 6), (17, 29, 16, 24))


def _threefry_bits(w):
    one = lambda c: jnp.asarray(c & 0xFFFFFFFF, w.dtype)
    x0 = w
    x1 = (lax.shift_left(w, one(13)) | lax.shift_right_logical(w, one(19))) ^ w
    for r in (15, 26, 6):
        x0 = x0 + x1
        x1 = (lax.shift_left(x1, one(r))
              | lax.shift_right_logical(x1, one(32 - r))) ^ x0
    x0 = x0 + one(42)
    x1 = x1 + one(_KS[2] + 1)
    for b in range(1, 5):
        for r in _ROT[b % 2]:
            x0 = x0 + x1
            x1 = (lax.shift_left(x1, one(r))
                  | lax.shift_right_logical(x1, one(32 - r))) ^ x0
        kx = _KS[(b + 1) % 3]
        if kx:
            x0 = x0 + one(kx)
        x1 = x1 + one(_KS[(b + 2) % 3] + b + 1)
    return x0 ^ x1


def _bits_to_gumbel(bits):
    bits = bits.astype(jnp.uint32)
    fb = (bits >> jnp.uint32(9)) | jnp.uint32(0x3F800000)
    f = lax.bitcast_convert_type(fb, jnp.float32) - jnp.float32(1.0)
    u = jnp.maximum(jnp.float32(1e-10), f + jnp.float32(1e-10))
    return -jnp.log(-jnp.log(u))


def _sc_bits():
    info = plsc.get_sparse_core_info()
    nw = info.num_cores * info.num_subcores
    t_per_w = N_TOK // nw
    n_chunks = t_per_w // CH
    mesh = plsc.VectorSubcoreMesh(core_axis_name="c", subcore_axis_name="s")

    @functools.partial(
        pl.kernel, mesh=mesh,
        compiler_params=pltpu.CompilerParams(use_tc_tiling_on_sc=True),
        out_type=jax.ShapeDtypeStruct((N_TOK, K_SC), jnp.int32),
        scratch_types=[
            pltpu.VMEM((CH, K_SC), jnp.int32),
            pltpu.VMEM((CH, K_SC), jnp.int32),
            pltpu.SemaphoreType.DMA,
            pltpu.SemaphoreType.DMA,
        ],
    )
    def k(bits_hbm, buf0, buf1, sem0, sem1):
        wid = lax.axis_index("s") * info.num_cores + lax.axis_index("c")
        t0 = wid * t_per_w
        lane = lax.iota(jnp.int32, 16)
        bufs = (buf0, buf1)
        sems = (sem0, sem1)

        def fill(c, buf):
            row0 = t0 + c * CH

            def row_body(r, _):
                base_r = (row0 + r) * N_EMB + K_TC + 42

                def vec(v8, _):
                    for u in range(8):
                        c16 = v8 * 8 + u
                        buf[r, pl.ds(c16 * 16, 16)] = _threefry_bits(
                            base_r + c16 * 16 + lane)
                    return 0

                lax.fori_loop(0, NVROW // 8, vec, 0)
                return 0

            lax.fori_loop(0, CH, row_body, 0)

        def fire(c, b):
            fill(c, bufs[b])
            pltpu.async_copy(
                bufs[b], bits_hbm.at[pl.ds(t0 + c * CH, CH), :], sems[b])

        def drain(c, b):
            pltpu.make_async_copy(
                bufs[b], bits_hbm.at[pl.ds(t0 + c * CH, CH), :], sems[b]).wait()

        fire(0, 0)
        fire(1, 1)

        def step(g, _):
            c0 = g * 2
            drain(c0 - 2, 0)
            fire(c0, 0)
            drain(c0 - 1, 1)
            fire(c0 + 1, 1)
            return 0

        lax.fori_loop(1, n_chunks // 2, step, 0)
        drain(n_chunks - 2, 0)
        drain(n_chunks - 1, 1)

    return k()


def _cnorm_body(c_ref, out_ref):
    c = c_ref[...]
    nrm = jnp.sqrt(jnp.sum(c * c, axis=-1, keepdims=True))
    out_ref[...] = c / jnp.maximum(nrm, jnp.float32(1e-8))


def _normalize_codebook(codebook):
    return pl.pallas_call(
        _cnorm_body,
        out_shape=jax.ShapeDtypeStruct((N_EMB, DIM), jnp.float32),
    )(codebook)


def _normalize_x(x_ref):
    x = x_ref[...]
    nrm = jnp.sqrt(jnp.sum(x * x, axis=-1, keepdims=True))
    return x / jnp.maximum(nrm, jnp.float32(1e-8))


def _head_body(x_ref, cn_ref, t_ref, bv_ref, bi_ref):
    i = pl.program_id(0)
    xn = _normalize_x(x_ref)
    temp = t_ref[0, 0]

    rows = i * TN + lax.broadcasted_iota(jnp.int32, (TN, TK), 0)
    cols = lax.broadcasted_iota(jnp.int32, (TN, TK), 1)
    w0 = (rows * N_EMB + cols + 42).astype(jnp.uint32)

    def kstep(j, carry):
        bv, bi = carry
        cn = cn_ref[pl.ds(j * TK, TK), :]
        l = lax.dot_general(
            xn, cn, (((1,), (1,)), ((), ())),
            preferred_element_type=jnp.float32,
        ) / temp
        w = w0 + jnp.uint32(TK) * j.astype(jnp.uint32)
        v = l + _bits_to_gumbel(_threefry_bits(w))
        mv = jnp.max(v, axis=1)
        cand = jnp.where(v == mv[:, None], cols, jnp.int32(TK))
        ma = jnp.min(cand, axis=1) + j * TK
        upd = mv > bv
        return jnp.where(upd, mv, bv), jnp.where(upd, ma, bi)

    bv0 = jnp.full((TN,), -jnp.inf, jnp.float32)
    bi0 = jnp.zeros((TN,), jnp.int32)
    bv, bi = lax.fori_loop(0, KB, kstep, (bv0, bi0))
    bv_ref[...] = bv.reshape(1, 1, TN)
    bi_ref[...] = bi.reshape(1, 1, TN)


def _tail_body(x_ref, cn_ref, t_ref, bits_ref, bv_ref, bi_ref, out_ref):
    xn = _normalize_x(x_ref)
    temp = t_ref[0, 0]
    cn = cn_ref[pl.ds(K_TC, K_SC), :]
    l = lax.dot_general(
        xn, cn, (((1,), (1,)), ((), ())),
        preferred_element_type=jnp.float32,
    ) / temp
    v = l + _bits_to_gumbel(bits_ref[...])
    mv = jnp.max(v, axis=1)
    cols = lax.broadcasted_iota(jnp.int32, (TN, K_SC), 1)
    cand = jnp.where(v == mv[:, None], cols, jnp.int32(K_SC))
    ma = jnp.min(cand, axis=1) + K_TC
    bv = bv_ref[...].reshape(TN)
    bi = bi_ref[...].reshape(TN)
    upd = mv > bv
    out_ref[...] = jnp.where(upd, ma, bi).reshape(1, 1, TN)


def _hard_indices(latent, cn, temperature, bits):
    t2 = temperature.reshape(1, 1)
    bv, bi = pl.pallas_call(
        _head_body,
        grid=(NB,),
        in_specs=[
            pl.BlockSpec((TN, DIM), lambda i: (i, 0)),
            pl.BlockSpec((N_EMB, DIM), lambda i: (0, 0)),
            pl.BlockSpec(memory_space=pltpu.SMEM),
        ],
        out_specs=[
            pl.BlockSpec((1, 1, TN), lambda i: (i, 0, 0)),
            pl.BlockSpec((1, 1, TN), lambda i: (i, 0, 0)),
        ],
        out_shape=[
            jax.ShapeDtypeStruct((NB, 1, TN), jnp.float32),
            jax.ShapeDtypeStruct((NB, 1, TN), jnp.int32),
        ],
    )(latent, cn, t2)
    out = pl.pallas_call(
        _tail_body,
        grid=(NB,),
        in_specs=[
            pl.BlockSpec((TN, DIM), lambda i: (i, 0)),
            pl.BlockSpec((N_EMB, DIM), lambda i: (0, 0)),
            pl.BlockSpec(memory_space=pltpu.SMEM),
            pl.BlockSpec((TN, K_SC), lambda i: (i, 0)),
            pl.BlockSpec((1, 1, TN), lambda i: (i, 0, 0)),
            pl.BlockSpec((1, 1, TN), lambda i: (i, 0, 0)),
        ],
        out_specs=pl.BlockSpec((1, 1, TN), lambda i: (i, 0, 0)),
        out_shape=jax.ShapeDtypeStruct((NB, 1, TN), jnp.int32),
    )(latent, cn, t2, bits, bv, bi)
    return out.reshape(N_TOK)


def _sc_gather(codebook, idx):
    info = plsc.get_sparse_core_info()
    nw = info.num_cores * info.num_subcores
    b_per_w = N_TOK // nw
    mesh = plsc.VectorSubcoreMesh(core_axis_name="c", subcore_axis_name="s")

    @functools.partial(
        pl.kernel, mesh=mesh,
        compiler_params=pltpu.CompilerParams(use_tc_tiling_on_sc=False),
        out_type=jax.ShapeDtypeStruct((N_TOK, DIM), jnp.float32),
        scratch_types=[
            pltpu.VMEM((b_per_w,), jnp.int32),
            pltpu.VMEM((b_per_w, DIM), jnp.float32),
            pltpu.SemaphoreType.DMA,
        ],
    )
    def k(table_hbm, idx_hbm, out_hbm, idx_v, rows_v, sem):
        wid = lax.axis_index("s") * info.num_cores + lax.axis_index("c")
        base = wid * b_per_w
        pltpu.sync_copy(idx_hbm.at[pl.ds(base, b_per_w)], idx_v)
        pltpu.async_copy(table_hbm.at[idx_v], rows_v, sem).wait()
        pltpu.sync_copy(rows_v, out_hbm.at[pl.ds(base, b_per_w)])

    return k(codebook, idx)


def kernel(latent_representation, codebook, temperature):
    bits = _sc_bits()
    cn = _normalize_codebook(codebook)
    idx = _hard_indices(latent_representation, cn, temperature, bits)
    return _sc_gather(codebook, idx)
